# baseline (speedup 1.0000x reference)
import functools

import jax
import jax.numpy as jnp
from jax import lax
from jax.experimental import pallas as pl
from jax.experimental.pallas import tpu as pltpu
from jax.experimental.pallas import tpu_sc as plsc

N = 10000
E = 160000
IN = 16
HID = 16
EMB = 16
EA = 16
NG = 64
D = 256

NC = 2
NS = 16
NW = NC * NS
CHUNK = 128
KPERW = 40
EPAD = NW * KPERW * CHUNK
EROWS = EPAD // CHUNK
NACC = 10240
STRIPE = NACC // NS
TRASH = N

_f32 = jnp.float32


def _leaky(v):
    return jnp.where(v >= 0, v, 0.01 * v)


def _sc_gather_cnt_body(src2d, dst2d, table, zeros, ones, xg_out, cnt_out,
                        idx_v, didx_v, rows_v, ones_v, acc_sh, sem):
    c = lax.axis_index("c")
    s = lax.axis_index("s")
    wid = s * NC + c
    pltpu.sync_copy(zeros.at[pl.ds(s * STRIPE, STRIPE)],
                    acc_sh.at[pl.ds(s * STRIPE, STRIPE)])
    pltpu.sync_copy(src2d.at[pl.ds(wid * KPERW, KPERW)], idx_v)
    pltpu.sync_copy(dst2d.at[pl.ds(wid * KPERW, KPERW)], didx_v)
    pltpu.sync_copy(ones, ones_v)
    plsc.subcore_barrier()

    def fire(j, carry):
        pltpu.async_copy(table.at[idx_v.at[j]], rows_v.at[j], sem)
        return carry
    lax.fori_loop(0, KPERW, fire, 0)

    def scat(j, carry):
        pltpu.sync_copy(ones_v, acc_sh.at[didx_v.at[j]], add=True)
        return carry
    lax.fori_loop(0, KPERW, scat, 0)

    def drain(j, carry):
        pltpu.make_async_copy(table.at[idx_v.at[j]], rows_v.at[j], sem).wait()
        return carry
    lax.fori_loop(0, KPERW, drain, 0)
    pltpu.sync_copy(rows_v, xg_out.at[pl.ds(wid * KPERW, KPERW)])

    plsc.subcore_barrier()
    pltpu.sync_copy(acc_sh.at[pl.ds(s * STRIPE, STRIPE)],
                    cnt_out.at[c, pl.ds(s * STRIPE, STRIPE)])


def _sc_gather_body(src2d, table, xg_out, idx_v, rows_v, sem):
    c = lax.axis_index("c")
    s = lax.axis_index("s")
    wid = s * NC + c
    pltpu.sync_copy(src2d.at[pl.ds(wid * KPERW, KPERW)], idx_v)

    def fire(j, carry):
        pltpu.async_copy(table.at[idx_v.at[j]], rows_v.at[j], sem)
        return carry
    lax.fori_loop(0, KPERW, fire, 0)

    def drain(j, carry):
        pltpu.make_async_copy(table.at[idx_v.at[j]], rows_v.at[j], sem).wait()
        return carry
    lax.fori_loop(0, KPERW, drain, 0)
    pltpu.sync_copy(rows_v, xg_out.at[pl.ds(wid * KPERW, KPERW)])


def _sc_scatter_body(dst2d, msg_in, zeros, acc_out, didx_v, rows_v, acc_sh):
    c = lax.axis_index("c")
    s = lax.axis_index("s")
    wid = s * NC + c
    pltpu.sync_copy(zeros.at[pl.ds(s * STRIPE, STRIPE)],
                    acc_sh.at[pl.ds(s * STRIPE, STRIPE)])
    pltpu.sync_copy(dst2d.at[pl.ds(wid * KPERW, KPERW)], didx_v)
    pltpu.sync_copy(msg_in.at[pl.ds(wid * KPERW, KPERW)], rows_v)
    plsc.subcore_barrier()

    def scat(j, carry):
        pltpu.sync_copy(rows_v.at[j], acc_sh.at[didx_v.at[j]], add=True)
        return carry
    lax.fori_loop(0, KPERW, scat, 0)

    plsc.subcore_barrier()
    pltpu.sync_copy(acc_sh.at[pl.ds(s * STRIPE, STRIPE)],
                    acc_out.at[c, pl.ds(s * STRIPE, STRIPE)])


@functools.cache
def _sc_kernels():
    mesh = plsc.VectorSubcoreMesh(
        core_axis_name="c", subcore_axis_name="s",
        num_cores=NC, num_subcores=NS)
    params = pltpu.CompilerParams(use_tc_tiling_on_sc=False)
    gather_cnt = pl.kernel(
        _sc_gather_cnt_body,
        out_type=[
            jax.ShapeDtypeStruct((EROWS, CHUNK, 16), _f32),
            jax.ShapeDtypeStruct((NC, NACC, 16), _f32),
        ],
        mesh=mesh,
        scratch_types=[
            pltpu.VMEM((KPERW, CHUNK), jnp.int32),
            pltpu.VMEM((KPERW, CHUNK), jnp.int32),
            pltpu.VMEM((KPERW, CHUNK, 16), _f32),
            pltpu.VMEM((CHUNK, 16), _f32),
            pltpu.VMEM_SHARED((NACC, 16), _f32),
            pltpu.SemaphoreType.DMA,
        ],
        compiler_params=params,
    )
    gather = pl.kernel(
        _sc_gather_body,
        out_type=jax.ShapeDtypeStruct((EROWS, CHUNK, 16), _f32),
        mesh=mesh,
        scratch_types=[
            pltpu.VMEM((KPERW, CHUNK), jnp.int32),
            pltpu.VMEM((KPERW, CHUNK, 16), _f32),
            pltpu.SemaphoreType.DMA,
        ],
        compiler_params=params,
    )
    scatter = pl.kernel(
        _sc_scatter_body,
        out_type=jax.ShapeDtypeStruct((NC, NACC, 16), _f32),
        mesh=mesh,
        scratch_types=[
            pltpu.VMEM((KPERW, CHUNK), jnp.int32),
            pltpu.VMEM((KPERW, CHUNK, 16), _f32),
            pltpu.VMEM_SHARED((NACC, 16), _f32),
        ],
        compiler_params=params,
    )
    return gather_cnt, gather, scatter


BE = 2048
BN = 2048


_HI = jax.lax.Precision.HIGHEST


def _edge_mlp_body(ea_ref, xg_ref, e1w_ref, e1b_ref, e2w_ref, e2b_ref,
                   out_ref):
    ht = jnp.dot(e1w_ref[...], ea_ref[...], preferred_element_type=_f32,
                 precision=_HI)
    ht = _leaky(ht + e1b_ref[...])
    wft = jnp.dot(e2w_ref[...], ht, preferred_element_type=_f32,
                  precision=_HI)
    wft = wft + e2b_ref[...]
    p = wft.reshape(16, 16, BE) * xg_ref[...][:, None, :]
    out_ref[...] = jnp.sum(p, axis=0)


def _edge_mlp(ea_t, xg_t, e1w_t, e1b_t, e2w_t, e2b_t):
    grid = EPAD // BE
    return pl.pallas_call(
        _edge_mlp_body,
        grid=(grid,),
        in_specs=[
            pl.BlockSpec((16, BE), lambda i: (0, i)),
            pl.BlockSpec((16, BE), lambda i: (0, i)),
            pl.BlockSpec((D, 16), lambda i: (0, 0)),
            pl.BlockSpec((D, 1), lambda i: (0, 0)),
            pl.BlockSpec((D, D), lambda i: (0, 0)),
            pl.BlockSpec((D, 1), lambda i: (0, 0)),
        ],
        out_specs=pl.BlockSpec((16, BE), lambda i: (0, i)),
        out_shape=jax.ShapeDtypeStruct((16, EPAD), _f32),
    )(ea_t, xg_t, e1w_t, e1b_t, e2w_t, e2b_t)


def _aggr_body(acc_ref, cnt_ref, x_ref, root_ref, bias_ref, out_ref, *, leaky):
    a = acc_ref[...]
    s = a[0] + a[1]
    ca = cnt_ref[...]
    cnt = ca[0, :, :1] + ca[1, :, :1]
    r = s / jnp.maximum(cnt, 1.0)
    r = r + jnp.dot(x_ref[...], root_ref[...], preferred_element_type=_f32,
                    precision=_HI)
    r = r + bias_ref[...]
    if leaky:
        r = _leaky(r)
    out_ref[...] = r


def _aggr(acc, cnt, x, root, bias, leaky):
    grid = NACC // BN
    return pl.pallas_call(
        functools.partial(_aggr_body, leaky=leaky),
        grid=(grid,),
        in_specs=[
            pl.BlockSpec((NC, BN, 16), lambda i: (0, i, 0)),
            pl.BlockSpec((NC, BN, 16), lambda i: (0, i, 0)),
            pl.BlockSpec((BN, 16), lambda i: (i, 0)),
            pl.BlockSpec((16, 16), lambda i: (0, 0)),
            pl.BlockSpec((1, 16), lambda i: (0, 0)),
        ],
        out_specs=pl.BlockSpec((BN, 16), lambda i: (i, 0)),
        out_shape=jax.ShapeDtypeStruct((NACC, 16), _f32),
    )(acc, cnt, x, root, bias)


def _pool_body(h_ref, w_ref, b_ref, day_ref, sec_ref, m1a_ref, m1d_ref,
               m1s_ref, m1b_ref, m2w_ref, m2b_ref, m3w_ref, m3b_ref,
               out_ref, acc_s, cnt_s):
    i = pl.program_id(0)

    @pl.when(i == 0)
    def _():
        acc_s[...] = jnp.zeros_like(acc_s)
        cnt_s[...] = jnp.zeros_like(cnt_s)

    oh = (b_ref[...] == lax.broadcasted_iota(jnp.int32, (BN, NG), 1))
    oh = oh.astype(_f32)
    w = w_ref[...]
    hv = h_ref[...] * w
    dn = (((0,), (0,)), ((), ()))
    acc_s[...] += lax.dot_general(oh, hv, dn, preferred_element_type=_f32,
                                  precision=_HI)
    cnt_s[...] += lax.dot_general(oh, w, dn, preferred_element_type=_f32,
                                  precision=_HI)

    @pl.when(i == (NACC // BN) - 1)
    def _():
        pooled = acc_s[...] / jnp.maximum(cnt_s[...], 1.0)
        z = jnp.dot(pooled, m1a_ref[...], preferred_element_type=_f32,
                    precision=_HI)
        z = z + day_ref[...] * m1d_ref[...] + sec_ref[...] * m1s_ref[...]
        z = _leaky(z + m1b_ref[...])
        z = _leaky(jnp.dot(z, m2w_ref[...], preferred_element_type=_f32,
                           precision=_HI) + m2b_ref[...])
        out_ref[...] = (jnp.dot(z, m3w_ref[...], preferred_element_type=_f32,
                                precision=_HI) + m3b_ref[...])


def _pool(h, w, b, day, sec, m1a, m1d, m1s, m1b, m2w, m2b, m3w, m3b):
    grid = NACC // BN
    return pl.pallas_call(
        _pool_body,
        grid=(grid,),
        in_specs=[
            pl.BlockSpec((BN, 16), lambda i: (i, 0)),
            pl.BlockSpec((BN, 1), lambda i: (i, 0)),
            pl.BlockSpec((BN, 1), lambda i: (i, 0)),
            pl.BlockSpec((NG, 1), lambda i: (0, 0)),
            pl.BlockSpec((NG, 1), lambda i: (0, 0)),
            pl.BlockSpec((16, 128), lambda i: (0, 0)),
            pl.BlockSpec((1, 128), lambda i: (0, 0)),
            pl.BlockSpec((1, 128), lambda i: (0, 0)),
            pl.BlockSpec((1, 128), lambda i: (0, 0)),
            pl.BlockSpec((128, 128), lambda i: (0, 0)),
            pl.BlockSpec((1, 128), lambda i: (0, 0)),
            pl.BlockSpec((128, 1), lambda i: (0, 0)),
            pl.BlockSpec((1, 1), lambda i: (0, 0)),
        ],
        out_specs=pl.BlockSpec((NG, 1), lambda i: (0, 0)),
        out_shape=jax.ShapeDtypeStruct((NG, 1), _f32),
        scratch_shapes=[
            pltpu.VMEM((NG, 16), _f32),
            pltpu.VMEM((NG, 1), _f32),
        ],
    )(h, w, b, day, sec, m1a, m1d, m1s, m1b, m2w, m2b, m3w, m3b)


@jax.jit
def kernel(x, edge_index, edge_attr, trip_mask, batch, day, sec,
           c1_e1w, c1_e1b, c1_e2w, c1_e2b, c1_root, c1_bias,
           c2_e1w, c2_e1b, c2_e2w, c2_e2b, c2_root, c2_bias,
           m1w, m1b, m2w, m2b, m3w, m3b):
    src = edge_index[0].astype(jnp.int32)
    dst = edge_index[1].astype(jnp.int32)
    src2d = jnp.concatenate(
        [src, jnp.zeros((EPAD - E,), jnp.int32)]).reshape(EROWS, CHUNK)
    dst2d = jnp.concatenate(
        [dst, jnp.full((EPAD - E,), TRASH, jnp.int32)]).reshape(EROWS, CHUNK)
    ea_p = jnp.concatenate(
        [edge_attr, jnp.zeros((EPAD - E, EA), _f32)], axis=0)
    x_p = jnp.concatenate([x, jnp.zeros((NACC - N, IN), _f32)], axis=0)
    zeros_acc = jnp.zeros((NACC, 16), _f32)
    ones_chunk = jnp.ones((CHUNK, 16), _f32)
    _gather_cnt, _gather, _scatter = _sc_kernels()

    ea_t = ea_p.T

    xg3, cntp = _gather_cnt(src2d, dst2d, x_p, zeros_acc, ones_chunk)
    msg1_t = _edge_mlp(ea_t, xg3.reshape(EPAD, 16).T,
                       c1_e1w.T, c1_e1b.reshape(D, 1), c1_e2w.T,
                       c1_e2b.reshape(D, 1))
    acc1 = _scatter(dst2d, msg1_t.T.reshape(EROWS, CHUNK, 16), zeros_acc)
    h1 = _aggr(acc1, cntp, x_p, c1_root, c1_bias.reshape(1, 16), leaky=True)

    hg3 = _gather(src2d, h1)
    msg2_t = _edge_mlp(ea_t, hg3.reshape(EPAD, 16).T,
                       c2_e1w.T, c2_e1b.reshape(D, 1), c2_e2w.T,
                       c2_e2b.reshape(D, 1))
    acc2 = _scatter(dst2d, msg2_t.T.reshape(EROWS, CHUNK, 16), zeros_acc)
    h2 = _aggr(acc2, cntp, h1, c2_root, c2_bias.reshape(1, 16), leaky=False)

    w_p = jnp.concatenate(
        [trip_mask.astype(_f32), jnp.zeros((NACC - N,), _f32)]).reshape(NACC, 1)
    b_p = jnp.concatenate(
        [batch.astype(jnp.int32),
         jnp.full((NACC - N,), NG, jnp.int32)]).reshape(NACC, 1)
    return _pool(h2, w_p, b_p, day.reshape(NG, 1), sec.reshape(NG, 1),
                 m1w[:16], m1w[16:17], m1w[17:18], m1b.reshape(1, 128),
                 m2w, m2b.reshape(1, 128), m3w, m3b.reshape(1, 1))


# speedup vs baseline: 1.3049x; 1.3049x over previous
import functools

import jax
import jax.numpy as jnp
from jax import lax
from jax.experimental import pallas as pl
from jax.experimental.pallas import tpu as pltpu
from jax.experimental.pallas import tpu_sc as plsc

N = 10000
E = 160000
IN = 16
HID = 16
EMB = 16
EA = 16
NG = 64
D = 256

NC = 2
NS = 16
NW = NC * NS
CHUNK = 128
KPERW = 40
EPAD = NW * KPERW * CHUNK
EROWS = EPAD // CHUNK
KH = KPERW // 2
EH = EPAD // 2
EHROWS = EROWS // 2
NACC = 10240
STRIPE = NACC // NS
TRASH = N

_f32 = jnp.float32


def _leaky(v):
    return jnp.where(v >= 0, v, 0.01 * v)


def _sc_gather_cnt_body(src2d, dst2d, table, zeros, ones, xg_out, cnt_out,
                        idx_v, didx_v, rows_v, ones_v, acc_sh, sem):
    c = lax.axis_index("c")
    s = lax.axis_index("s")
    wid = s * NC + c
    pltpu.sync_copy(zeros.at[pl.ds(s * STRIPE, STRIPE)],
                    acc_sh.at[pl.ds(s * STRIPE, STRIPE)])
    pltpu.sync_copy(src2d.at[pl.ds(wid * KH, KH)], idx_v)
    pltpu.sync_copy(dst2d.at[pl.ds(wid * KH, KH)], didx_v)
    pltpu.sync_copy(ones, ones_v)
    plsc.subcore_barrier()

    def fire(j, carry):
        pltpu.async_copy(table.at[idx_v.at[j]], rows_v.at[j], sem)
        return carry
    lax.fori_loop(0, KH, fire, 0)

    def scat(j, carry):
        pltpu.sync_copy(ones_v, acc_sh.at[didx_v.at[j]], add=True)
        return carry
    lax.fori_loop(0, KH, scat, 0)

    def drain(j, carry):
        pltpu.make_async_copy(table.at[idx_v.at[j]], rows_v.at[j], sem).wait()
        return carry
    lax.fori_loop(0, KH, drain, 0)
    pltpu.sync_copy(rows_v, xg_out.at[pl.ds(wid * KH, KH)])

    plsc.subcore_barrier()
    pltpu.sync_copy(acc_sh.at[pl.ds(s * STRIPE, STRIPE)],
                    cnt_out.at[c, pl.ds(s * STRIPE, STRIPE)])


def _sc_gather_body(src2d, table, xg_out, idx_v, rows_v, sem):
    c = lax.axis_index("c")
    s = lax.axis_index("s")
    wid = s * NC + c
    pltpu.sync_copy(src2d.at[pl.ds(wid * KH, KH)], idx_v)

    def fire(j, carry):
        pltpu.async_copy(table.at[idx_v.at[j]], rows_v.at[j], sem)
        return carry
    lax.fori_loop(0, KH, fire, 0)

    def drain(j, carry):
        pltpu.make_async_copy(table.at[idx_v.at[j]], rows_v.at[j], sem).wait()
        return carry
    lax.fori_loop(0, KH, drain, 0)
    pltpu.sync_copy(rows_v, xg_out.at[pl.ds(wid * KH, KH)])


def _sc_scatter_body(dst2d, msg_in, zeros, acc_out, didx_v, rows_v, acc_sh):
    c = lax.axis_index("c")
    s = lax.axis_index("s")
    wid = s * NC + c
    pltpu.sync_copy(zeros.at[pl.ds(s * STRIPE, STRIPE)],
                    acc_sh.at[pl.ds(s * STRIPE, STRIPE)])
    pltpu.sync_copy(dst2d.at[pl.ds(wid * KH, KH)], didx_v)
    pltpu.sync_copy(msg_in.at[pl.ds(wid * KH, KH)], rows_v)
    plsc.subcore_barrier()

    def scat(j, carry):
        pltpu.sync_copy(rows_v.at[j], acc_sh.at[didx_v.at[j]], add=True)
        return carry
    lax.fori_loop(0, KH, scat, 0)

    plsc.subcore_barrier()
    pltpu.sync_copy(acc_sh.at[pl.ds(s * STRIPE, STRIPE)],
                    acc_out.at[c, pl.ds(s * STRIPE, STRIPE)])


@functools.cache
def _sc_kernels():
    mesh = plsc.VectorSubcoreMesh(
        core_axis_name="c", subcore_axis_name="s",
        num_cores=NC, num_subcores=NS)
    params = pltpu.CompilerParams(use_tc_tiling_on_sc=False)
    gather_cnt = pl.kernel(
        _sc_gather_cnt_body,
        out_type=[
            jax.ShapeDtypeStruct((EHROWS, CHUNK, 16), _f32),
            jax.ShapeDtypeStruct((NC, NACC, 16), _f32),
        ],
        mesh=mesh,
        scratch_types=[
            pltpu.VMEM((KH, CHUNK), jnp.int32),
            pltpu.VMEM((KH, CHUNK), jnp.int32),
            pltpu.VMEM((KH, CHUNK, 16), _f32),
            pltpu.VMEM((CHUNK, 16), _f32),
            pltpu.VMEM_SHARED((NACC, 16), _f32),
            pltpu.SemaphoreType.DMA,
        ],
        compiler_params=params,
    )
    gather = pl.kernel(
        _sc_gather_body,
        out_type=jax.ShapeDtypeStruct((EHROWS, CHUNK, 16), _f32),
        mesh=mesh,
        scratch_types=[
            pltpu.VMEM((KH, CHUNK), jnp.int32),
            pltpu.VMEM((KH, CHUNK, 16), _f32),
            pltpu.SemaphoreType.DMA,
        ],
        compiler_params=params,
    )
    scatter = pl.kernel(
        _sc_scatter_body,
        out_type=jax.ShapeDtypeStruct((NC, NACC, 16), _f32),
        mesh=mesh,
        scratch_types=[
            pltpu.VMEM((KH, CHUNK), jnp.int32),
            pltpu.VMEM((KH, CHUNK, 16), _f32),
            pltpu.VMEM_SHARED((NACC, 16), _f32),
        ],
        compiler_params=params,
    )
    return gather_cnt, gather, scatter


BE = 2048
BN = 2048


_HI = jax.lax.Precision.HIGHEST


def _edge_mlp_body(ea_ref, xg_ref, e1w_ref, e1b_ref, e2w_ref, e2b_ref,
                   out_ref):
    ht = jnp.dot(e1w_ref[...], ea_ref[...], preferred_element_type=_f32)
    ht = _leaky(ht + e1b_ref[...])
    wft = jnp.dot(e2w_ref[...], ht, preferred_element_type=_f32)
    wft = wft + e2b_ref[...]
    p = wft.reshape(16, 16, BE) * xg_ref[...][:, None, :]
    out_ref[...] = jnp.sum(p, axis=0)


def _edge_mlp(ea_t, xg_t, e1w_t, e1b_t, e2w_t, e2b_t, off):
    grid = EH // BE
    return pl.pallas_call(
        _edge_mlp_body,
        grid=(grid,),
        in_specs=[
            pl.BlockSpec((16, BE), lambda i, off=off: (0, i + off)),
            pl.BlockSpec((16, BE), lambda i: (0, i)),
            pl.BlockSpec((D, 16), lambda i: (0, 0)),
            pl.BlockSpec((D, 1), lambda i: (0, 0)),
            pl.BlockSpec((D, D), lambda i: (0, 0)),
            pl.BlockSpec((D, 1), lambda i: (0, 0)),
        ],
        out_specs=pl.BlockSpec((16, BE), lambda i: (0, i)),
        out_shape=jax.ShapeDtypeStruct((16, EH), _f32),
    )(ea_t, xg_t, e1w_t, e1b_t, e2w_t, e2b_t)


def _aggr_body(acc_ref, cnt_ref, x_ref, root_ref, bias_ref, out_ref, *, leaky):
    a = acc_ref[...]
    s = a[0] + a[1] + a[2] + a[3]
    ca = cnt_ref[...]
    cnt = (ca[0, :, :1] + ca[1, :, :1] + ca[2, :, :1] + ca[3, :, :1])
    r = s / jnp.maximum(cnt, 1.0)
    r = r + jnp.dot(x_ref[...], root_ref[...], preferred_element_type=_f32)
    r = r + bias_ref[...]
    if leaky:
        r = _leaky(r)
    out_ref[...] = r


def _aggr(acc, cnt, x, root, bias, leaky):
    grid = NACC // BN
    return pl.pallas_call(
        functools.partial(_aggr_body, leaky=leaky),
        grid=(grid,),
        in_specs=[
            pl.BlockSpec((2 * NC, BN, 16), lambda i: (0, i, 0)),
            pl.BlockSpec((2 * NC, BN, 16), lambda i: (0, i, 0)),
            pl.BlockSpec((BN, 16), lambda i: (i, 0)),
            pl.BlockSpec((16, 16), lambda i: (0, 0)),
            pl.BlockSpec((1, 16), lambda i: (0, 0)),
        ],
        out_specs=pl.BlockSpec((BN, 16), lambda i: (i, 0)),
        out_shape=jax.ShapeDtypeStruct((NACC, 16), _f32),
    )(acc, cnt, x, root, bias)


def _pool_body(h_ref, w_ref, b_ref, day_ref, sec_ref, m1a_ref, m1d_ref,
               m1s_ref, m1b_ref, m2w_ref, m2b_ref, m3w_ref, m3b_ref,
               out_ref, acc_s, cnt_s):
    i = pl.program_id(0)

    @pl.when(i == 0)
    def _():
        acc_s[...] = jnp.zeros_like(acc_s)
        cnt_s[...] = jnp.zeros_like(cnt_s)

    oh = (b_ref[...] == lax.broadcasted_iota(jnp.int32, (BN, NG), 1))
    oh = oh.astype(_f32)
    w = w_ref[...]
    hv = h_ref[...] * w
    dn = (((0,), (0,)), ((), ()))
    acc_s[...] += lax.dot_general(oh, hv, dn, preferred_element_type=_f32,
                                  precision=_HI)
    cnt_s[...] += lax.dot_general(oh, w, dn, preferred_element_type=_f32,
                                  precision=_HI)

    @pl.when(i == (NACC // BN) - 1)
    def _():
        pooled = acc_s[...] / jnp.maximum(cnt_s[...], 1.0)
        z = jnp.dot(pooled, m1a_ref[...], preferred_element_type=_f32)
        z = z + day_ref[...] * m1d_ref[...] + sec_ref[...] * m1s_ref[...]
        z = _leaky(z + m1b_ref[...])
        z = _leaky(jnp.dot(z, m2w_ref[...], preferred_element_type=_f32)
                   + m2b_ref[...])
        out_ref[...] = (jnp.dot(z, m3w_ref[...], preferred_element_type=_f32)
                        + m3b_ref[...])


def _pool(h, w, b, day, sec, m1a, m1d, m1s, m1b, m2w, m2b, m3w, m3b):
    grid = NACC // BN
    return pl.pallas_call(
        _pool_body,
        grid=(grid,),
        in_specs=[
            pl.BlockSpec((BN, 16), lambda i: (i, 0)),
            pl.BlockSpec((BN, 1), lambda i: (i, 0)),
            pl.BlockSpec((BN, 1), lambda i: (i, 0)),
            pl.BlockSpec((NG, 1), lambda i: (0, 0)),
            pl.BlockSpec((NG, 1), lambda i: (0, 0)),
            pl.BlockSpec((16, 128), lambda i: (0, 0)),
            pl.BlockSpec((1, 128), lambda i: (0, 0)),
            pl.BlockSpec((1, 128), lambda i: (0, 0)),
            pl.BlockSpec((1, 128), lambda i: (0, 0)),
            pl.BlockSpec((128, 128), lambda i: (0, 0)),
            pl.BlockSpec((1, 128), lambda i: (0, 0)),
            pl.BlockSpec((128, 1), lambda i: (0, 0)),
            pl.BlockSpec((1, 1), lambda i: (0, 0)),
        ],
        out_specs=pl.BlockSpec((NG, 1), lambda i: (0, 0)),
        out_shape=jax.ShapeDtypeStruct((NG, 1), _f32),
        scratch_shapes=[
            pltpu.VMEM((NG, 16), _f32),
            pltpu.VMEM((NG, 1), _f32),
        ],
    )(h, w, b, day, sec, m1a, m1d, m1s, m1b, m2w, m2b, m3w, m3b)


@jax.jit
def kernel(x, edge_index, edge_attr, trip_mask, batch, day, sec,
           c1_e1w, c1_e1b, c1_e2w, c1_e2b, c1_root, c1_bias,
           c2_e1w, c2_e1b, c2_e2w, c2_e2b, c2_root, c2_bias,
           m1w, m1b, m2w, m2b, m3w, m3b):
    src = edge_index[0].astype(jnp.int32)
    dst = edge_index[1].astype(jnp.int32)
    src2d = jnp.concatenate(
        [src, jnp.zeros((EPAD - E,), jnp.int32)]).reshape(EROWS, CHUNK)
    dst2d = jnp.concatenate(
        [dst, jnp.full((EPAD - E,), TRASH, jnp.int32)]).reshape(EROWS, CHUNK)
    ea_p = jnp.concatenate(
        [edge_attr, jnp.zeros((EPAD - E, EA), _f32)], axis=0)
    x_p = jnp.concatenate([x, jnp.zeros((NACC - N, IN), _f32)], axis=0)
    zeros_acc = jnp.zeros((NACC, 16), _f32)
    ones_chunk = jnp.ones((CHUNK, 16), _f32)
    _gather_cnt, _gather, _scatter = _sc_kernels()

    ea_t = ea_p.T
    srcA, srcB = src2d[:EHROWS], src2d[EHROWS:]
    dstA, dstB = dst2d[:EHROWS], dst2d[EHROWS:]
    GH = EH // BE
    w1 = (c1_e1w.T, c1_e1b.reshape(D, 1), c1_e2w.T, c1_e2b.reshape(D, 1))
    w2 = (c2_e1w.T, c2_e1b.reshape(D, 1), c2_e2w.T, c2_e2b.reshape(D, 1))

    xgA, cntA = _gather_cnt(srcA, dstA, x_p, zeros_acc, ones_chunk)
    xgB, cntB = _gather_cnt(srcB, dstB, x_p, zeros_acc, ones_chunk)
    msgA = _edge_mlp(ea_t, xgA.reshape(EH, 16).T, *w1, off=0)
    msgB = _edge_mlp(ea_t, xgB.reshape(EH, 16).T, *w1, off=GH)
    accA = _scatter(dstA, msgA.T.reshape(EHROWS, CHUNK, 16), zeros_acc)
    accB = _scatter(dstB, msgB.T.reshape(EHROWS, CHUNK, 16), zeros_acc)
    cnt4 = jnp.concatenate([cntA, cntB], axis=0)
    acc4 = jnp.concatenate([accA, accB], axis=0)
    h1 = _aggr(acc4, cnt4, x_p, c1_root, c1_bias.reshape(1, 16), leaky=True)

    hgA = _gather(srcA, h1)
    hgB = _gather(srcB, h1)
    msg2A = _edge_mlp(ea_t, hgA.reshape(EH, 16).T, *w2, off=0)
    msg2B = _edge_mlp(ea_t, hgB.reshape(EH, 16).T, *w2, off=GH)
    acc2A = _scatter(dstA, msg2A.T.reshape(EHROWS, CHUNK, 16), zeros_acc)
    acc2B = _scatter(dstB, msg2B.T.reshape(EHROWS, CHUNK, 16), zeros_acc)
    acc24 = jnp.concatenate([acc2A, acc2B], axis=0)
    h2 = _aggr(acc24, cnt4, h1, c2_root, c2_bias.reshape(1, 16), leaky=False)

    w_p = jnp.concatenate(
        [trip_mask.astype(_f32), jnp.zeros((NACC - N,), _f32)]).reshape(NACC, 1)
    b_p = jnp.concatenate(
        [batch.astype(jnp.int32),
         jnp.full((NACC - N,), NG, jnp.int32)]).reshape(NACC, 1)
    return _pool(h2, w_p, b_p, day.reshape(NG, 1), sec.reshape(NG, 1),
                 m1w[:16], m1w[16:17], m1w[17:18], m1b.reshape(1, 128),
                 m2w, m2b.reshape(1, 128), m3w, m3b.reshape(1, 1))


# speedup vs baseline: 1.6336x; 1.2519x over previous
import functools

import jax
import jax.numpy as jnp
from jax import lax
from jax.experimental import pallas as pl
from jax.experimental.pallas import tpu as pltpu
from jax.experimental.pallas import tpu_sc as plsc

N = 10000
E = 160000
IN = 16
HID = 16
EMB = 16
EA = 16
NG = 64
D = 256

NC = 2
NS = 16
NW = NC * NS
CHUNK = 128
KPERW = 40
EPAD = NW * KPERW * CHUNK
EROWS = EPAD // CHUNK
KH = KPERW // 2
EH = EPAD // 2
EHROWS = EROWS // 2
NACC = 10240
STRIPE = NACC // NS
TRASH = N

_f32 = jnp.float32


def _leaky(v):
    return jnp.where(v >= 0, v, 0.01 * v)


def _sc_gather_cnt_body(src2d, dst2d, table, zeros, ones, xg_out, cnt_out,
                        idx_v, didx_v, rows_v, ones_v, acc_sh, sem, ssem):
    c = lax.axis_index("c")
    s = lax.axis_index("s")
    wid = s * NC + c
    pltpu.sync_copy(zeros.at[pl.ds(s * STRIPE, STRIPE)],
                    acc_sh.at[pl.ds(s * STRIPE, STRIPE)])
    pltpu.sync_copy(src2d.at[pl.ds(wid * KPERW, KPERW)], idx_v)
    pltpu.sync_copy(dst2d.at[pl.ds(wid * KPERW, KPERW)], didx_v)
    pltpu.sync_copy(ones, ones_v)
    plsc.subcore_barrier()

    gds = [pltpu.async_copy(table.at[idx_v.at[j]], rows_v.at[j], sem)
           for j in range(KPERW)]
    sds = [pltpu.async_copy(ones_v, acc_sh.at[didx_v.at[j]], ssem, add=True)
           for j in range(KPERW)]
    for d in gds:
        d.wait()
    pltpu.sync_copy(rows_v, xg_out.at[pl.ds(wid * KPERW, KPERW)])
    for d in sds:
        d.wait()

    plsc.subcore_barrier()
    pltpu.sync_copy(acc_sh.at[pl.ds(s * STRIPE, STRIPE)],
                    cnt_out.at[c, pl.ds(s * STRIPE, STRIPE)])


def _sc_gather_body(src2d, table, xg_out, idx_v, rows_v, sem):
    c = lax.axis_index("c")
    s = lax.axis_index("s")
    wid = s * NC + c
    pltpu.sync_copy(src2d.at[pl.ds(wid * KPERW, KPERW)], idx_v)

    gds = [pltpu.async_copy(table.at[idx_v.at[j]], rows_v.at[j], sem)
           for j in range(KPERW)]
    for d in gds:
        d.wait()
    pltpu.sync_copy(rows_v, xg_out.at[pl.ds(wid * KPERW, KPERW)])


def _sc_scatter_body(dst2d, msg_in, zeros, acc_out, didx_v, rows_v, acc_sh,
                     ssem):
    c = lax.axis_index("c")
    s = lax.axis_index("s")
    wid = s * NC + c
    pltpu.sync_copy(zeros.at[pl.ds(s * STRIPE, STRIPE)],
                    acc_sh.at[pl.ds(s * STRIPE, STRIPE)])
    pltpu.sync_copy(dst2d.at[pl.ds(wid * KPERW, KPERW)], didx_v)
    pltpu.sync_copy(msg_in.at[pl.ds(wid * KPERW, KPERW)], rows_v)
    plsc.subcore_barrier()

    sds = [pltpu.async_copy(rows_v.at[j], acc_sh.at[didx_v.at[j]], ssem,
                            add=True)
           for j in range(KPERW)]
    for d in sds:
        d.wait()

    plsc.subcore_barrier()
    pltpu.sync_copy(acc_sh.at[pl.ds(s * STRIPE, STRIPE)],
                    acc_out.at[c, pl.ds(s * STRIPE, STRIPE)])


@functools.cache
def _sc_kernels():
    mesh = plsc.VectorSubcoreMesh(
        core_axis_name="c", subcore_axis_name="s",
        num_cores=NC, num_subcores=NS)
    params = pltpu.CompilerParams(use_tc_tiling_on_sc=False)
    gather_cnt = pl.kernel(
        _sc_gather_cnt_body,
        out_type=[
            jax.ShapeDtypeStruct((EROWS, CHUNK, 16), _f32),
            jax.ShapeDtypeStruct((NC, NACC, 16), _f32),
        ],
        mesh=mesh,
        scratch_types=[
            pltpu.VMEM((KPERW, CHUNK), jnp.int32),
            pltpu.VMEM((KPERW, CHUNK), jnp.int32),
            pltpu.VMEM((KPERW, CHUNK, 16), _f32),
            pltpu.VMEM((CHUNK, 16), _f32),
            pltpu.VMEM_SHARED((NACC, 16), _f32),
            pltpu.SemaphoreType.DMA,
            pltpu.SemaphoreType.DMA,
        ],
        compiler_params=params,
    )
    gather = pl.kernel(
        _sc_gather_body,
        out_type=jax.ShapeDtypeStruct((EROWS, CHUNK, 16), _f32),
        mesh=mesh,
        scratch_types=[
            pltpu.VMEM((KPERW, CHUNK), jnp.int32),
            pltpu.VMEM((KPERW, CHUNK, 16), _f32),
            pltpu.SemaphoreType.DMA,
        ],
        compiler_params=params,
    )
    scatter = pl.kernel(
        _sc_scatter_body,
        out_type=jax.ShapeDtypeStruct((NC, NACC, 16), _f32),
        mesh=mesh,
        scratch_types=[
            pltpu.VMEM((KPERW, CHUNK), jnp.int32),
            pltpu.VMEM((KPERW, CHUNK, 16), _f32),
            pltpu.VMEM_SHARED((NACC, 16), _f32),
            pltpu.SemaphoreType.DMA,
        ],
        compiler_params=params,
    )
    return gather_cnt, gather, scatter


BE = 2048
BN = 2048


_HI = jax.lax.Precision.HIGHEST


def _edge_mlp_body(ea_ref, xg_ref, e1w_ref, e1b_ref, e2w_ref, e2b_ref,
                   out_ref):
    ht = jnp.dot(e1w_ref[...], ea_ref[...], preferred_element_type=_f32)
    ht = _leaky(ht + e1b_ref[...])
    wft = jnp.dot(e2w_ref[...], ht, preferred_element_type=_f32)
    wft = wft + e2b_ref[...]
    p = wft.reshape(16, 16, BE) * xg_ref[...][:, None, :]
    out_ref[...] = jnp.sum(p, axis=0)


def _edge_mlp(ea_t, xg_t, e1w_t, e1b_t, e2w_t, e2b_t):
    grid = EPAD // BE
    return pl.pallas_call(
        _edge_mlp_body,
        grid=(grid,),
        in_specs=[
            pl.BlockSpec((16, BE), lambda i: (0, i)),
            pl.BlockSpec((16, BE), lambda i: (0, i)),
            pl.BlockSpec((D, 16), lambda i: (0, 0)),
            pl.BlockSpec((D, 1), lambda i: (0, 0)),
            pl.BlockSpec((D, D), lambda i: (0, 0)),
            pl.BlockSpec((D, 1), lambda i: (0, 0)),
        ],
        out_specs=pl.BlockSpec((16, BE), lambda i: (0, i)),
        out_shape=jax.ShapeDtypeStruct((16, EPAD), _f32),
    )(ea_t, xg_t, e1w_t, e1b_t, e2w_t, e2b_t)


def _aggr_body(acc_ref, cnt_ref, x_ref, root_ref, bias_ref, out_ref, *, leaky):
    a = acc_ref[...]
    s = a[0] + a[1]
    ca = cnt_ref[...]
    cnt = ca[0, :, :1] + ca[1, :, :1]
    r = s / jnp.maximum(cnt, 1.0)
    r = r + jnp.dot(x_ref[...], root_ref[...], preferred_element_type=_f32)
    r = r + bias_ref[...]
    if leaky:
        r = _leaky(r)
    out_ref[...] = r


def _aggr(acc, cnt, x, root, bias, leaky):
    grid = NACC // BN
    return pl.pallas_call(
        functools.partial(_aggr_body, leaky=leaky),
        grid=(grid,),
        in_specs=[
            pl.BlockSpec((NC, BN, 16), lambda i: (0, i, 0)),
            pl.BlockSpec((NC, BN, 16), lambda i: (0, i, 0)),
            pl.BlockSpec((BN, 16), lambda i: (i, 0)),
            pl.BlockSpec((16, 16), lambda i: (0, 0)),
            pl.BlockSpec((1, 16), lambda i: (0, 0)),
        ],
        out_specs=pl.BlockSpec((BN, 16), lambda i: (i, 0)),
        out_shape=jax.ShapeDtypeStruct((NACC, 16), _f32),
    )(acc, cnt, x, root, bias)


def _pool_body(h_ref, w_ref, b_ref, day_ref, sec_ref, m1a_ref, m1d_ref,
               m1s_ref, m1b_ref, m2w_ref, m2b_ref, m3w_ref, m3b_ref,
               out_ref, acc_s, cnt_s):
    i = pl.program_id(0)

    @pl.when(i == 0)
    def _():
        acc_s[...] = jnp.zeros_like(acc_s)
        cnt_s[...] = jnp.zeros_like(cnt_s)

    oh = (b_ref[...] == lax.broadcasted_iota(jnp.int32, (BN, NG), 1))
    oh = oh.astype(_f32)
    w = w_ref[...]
    hv = h_ref[...] * w
    dn = (((0,), (0,)), ((), ()))
    acc_s[...] += lax.dot_general(oh, hv, dn, preferred_element_type=_f32,
                                  precision=_HI)
    cnt_s[...] += lax.dot_general(oh, w, dn, preferred_element_type=_f32,
                                  precision=_HI)

    @pl.when(i == (NACC // BN) - 1)
    def _():
        pooled = acc_s[...] / jnp.maximum(cnt_s[...], 1.0)
        z = jnp.dot(pooled, m1a_ref[...], preferred_element_type=_f32)
        z = z + day_ref[...] * m1d_ref[...] + sec_ref[...] * m1s_ref[...]
        z = _leaky(z + m1b_ref[...])
        z = _leaky(jnp.dot(z, m2w_ref[...], preferred_element_type=_f32)
                   + m2b_ref[...])
        out_ref[...] = (jnp.dot(z, m3w_ref[...], preferred_element_type=_f32)
                        + m3b_ref[...])


def _pool(h, w, b, day, sec, m1a, m1d, m1s, m1b, m2w, m2b, m3w, m3b):
    grid = NACC // BN
    return pl.pallas_call(
        _pool_body,
        grid=(grid,),
        in_specs=[
            pl.BlockSpec((BN, 16), lambda i: (i, 0)),
            pl.BlockSpec((BN, 1), lambda i: (i, 0)),
            pl.BlockSpec((BN, 1), lambda i: (i, 0)),
            pl.BlockSpec((NG, 1), lambda i: (0, 0)),
            pl.BlockSpec((NG, 1), lambda i: (0, 0)),
            pl.BlockSpec((16, 128), lambda i: (0, 0)),
            pl.BlockSpec((1, 128), lambda i: (0, 0)),
            pl.BlockSpec((1, 128), lambda i: (0, 0)),
            pl.BlockSpec((1, 128), lambda i: (0, 0)),
            pl.BlockSpec((128, 128), lambda i: (0, 0)),
            pl.BlockSpec((1, 128), lambda i: (0, 0)),
            pl.BlockSpec((128, 1), lambda i: (0, 0)),
            pl.BlockSpec((1, 1), lambda i: (0, 0)),
        ],
        out_specs=pl.BlockSpec((NG, 1), lambda i: (0, 0)),
        out_shape=jax.ShapeDtypeStruct((NG, 1), _f32),
        scratch_shapes=[
            pltpu.VMEM((NG, 16), _f32),
            pltpu.VMEM((NG, 1), _f32),
        ],
    )(h, w, b, day, sec, m1a, m1d, m1s, m1b, m2w, m2b, m3w, m3b)


@jax.jit
def kernel(x, edge_index, edge_attr, trip_mask, batch, day, sec,
           c1_e1w, c1_e1b, c1_e2w, c1_e2b, c1_root, c1_bias,
           c2_e1w, c2_e1b, c2_e2w, c2_e2b, c2_root, c2_bias,
           m1w, m1b, m2w, m2b, m3w, m3b):
    src = edge_index[0].astype(jnp.int32)
    dst = edge_index[1].astype(jnp.int32)
    src2d = jnp.concatenate(
        [src, jnp.zeros((EPAD - E,), jnp.int32)]).reshape(EROWS, CHUNK)
    dst2d = jnp.concatenate(
        [dst, jnp.full((EPAD - E,), TRASH, jnp.int32)]).reshape(EROWS, CHUNK)
    ea_p = jnp.concatenate(
        [edge_attr, jnp.zeros((EPAD - E, EA), _f32)], axis=0)
    x_p = jnp.concatenate([x, jnp.zeros((NACC - N, IN), _f32)], axis=0)
    zeros_acc = jnp.zeros((NACC, 16), _f32)
    ones_chunk = jnp.ones((CHUNK, 16), _f32)
    _gather_cnt, _gather, _scatter = _sc_kernels()

    ea_t = ea_p.T

    xg3, cntp = _gather_cnt(src2d, dst2d, x_p, zeros_acc, ones_chunk)
    msg1_t = _edge_mlp(ea_t, xg3.reshape(EPAD, 16).T,
                       c1_e1w.T, c1_e1b.reshape(D, 1), c1_e2w.T,
                       c1_e2b.reshape(D, 1))
    acc1 = _scatter(dst2d, msg1_t.T.reshape(EROWS, CHUNK, 16), zeros_acc)
    h1 = _aggr(acc1, cntp, x_p, c1_root, c1_bias.reshape(1, 16), leaky=True)

    hg3 = _gather(src2d, h1)
    msg2_t = _edge_mlp(ea_t, hg3.reshape(EPAD, 16).T,
                       c2_e1w.T, c2_e1b.reshape(D, 1), c2_e2w.T,
                       c2_e2b.reshape(D, 1))
    acc2 = _scatter(dst2d, msg2_t.T.reshape(EROWS, CHUNK, 16), zeros_acc)
    h2 = _aggr(acc2, cntp, h1, c2_root, c2_bias.reshape(1, 16), leaky=False)

    w_p = jnp.concatenate(
        [trip_mask.astype(_f32), jnp.zeros((NACC - N,), _f32)]).reshape(NACC, 1)
    b_p = jnp.concatenate(
        [batch.astype(jnp.int32),
         jnp.full((NACC - N,), NG, jnp.int32)]).reshape(NACC, 1)
    return _pool(h2, w_p, b_p, day.reshape(NG, 1), sec.reshape(NG, 1),
                 m1w[:16], m1w[16:17], m1w[17:18], m1b.reshape(1, 128),
                 m2w, m2b.reshape(1, 128), m3w, m3b.reshape(1, 1))


# speedup vs baseline: 1.6491x; 1.0094x over previous
import functools

import jax
import jax.numpy as jnp
from jax import lax
from jax.experimental import pallas as pl
from jax.experimental.pallas import tpu as pltpu
from jax.experimental.pallas import tpu_sc as plsc

N = 10000
E = 160000
IN = 16
HID = 16
EMB = 16
EA = 16
NG = 64
D = 256

NC = 2
NS = 16
NW = NC * NS
CHUNK = 128
KPERW = 40
EPAD = NW * KPERW * CHUNK
EROWS = EPAD // CHUNK
KH = KPERW // 2
EH = EPAD // 2
EHROWS = EROWS // 2
NACC = 10240
STRIPE = NACC // NS
TRASH = N

_f32 = jnp.float32


def _leaky(v):
    return jnp.where(v >= 0, v, 0.01 * v)


def _sc_gather_cnt_body(src2d, dst2d, table, zeros, ones, xg_out, cnt_out,
                        idx_v, didx_v, rows_v, ones_v, acc_sh, sem):
    c = lax.axis_index("c")
    s = lax.axis_index("s")
    wid = s * NC + c
    pltpu.sync_copy(zeros.at[pl.ds(s * STRIPE, STRIPE)],
                    acc_sh.at[pl.ds(s * STRIPE, STRIPE)])
    pltpu.sync_copy(src2d.at[pl.ds(wid * KPERW, KPERW)], idx_v)
    pltpu.sync_copy(dst2d.at[pl.ds(wid * KPERW, KPERW)], didx_v)
    pltpu.sync_copy(ones, ones_v)
    plsc.subcore_barrier()

    def fire(j, carry):
        pltpu.async_copy(table.at[idx_v.at[j]], rows_v.at[j], sem)
        return carry
    lax.fori_loop(0, KPERW, fire, 0)

    def scat(j, carry):
        pltpu.sync_copy(ones_v, acc_sh.at[didx_v.at[j]], add=True)
        return carry
    lax.fori_loop(0, KPERW, scat, 0)

    def drain(j, carry):
        pltpu.make_async_copy(table.at[idx_v.at[j]], rows_v.at[j], sem).wait()
        return carry
    lax.fori_loop(0, KPERW, drain, 0)
    pltpu.sync_copy(rows_v, xg_out.at[pl.ds(wid * KPERW, KPERW)])

    plsc.subcore_barrier()
    pltpu.sync_copy(acc_sh.at[pl.ds(s * STRIPE, STRIPE)],
                    cnt_out.at[c, pl.ds(s * STRIPE, STRIPE)])


def _sc_gather_body(src2d, table, xg_out, idx_v, rows_v, sem):
    c = lax.axis_index("c")
    s = lax.axis_index("s")
    wid = s * NC + c
    pltpu.sync_copy(src2d.at[pl.ds(wid * KPERW, KPERW)], idx_v)

    def fire(j, carry):
        pltpu.async_copy(table.at[idx_v.at[j]], rows_v.at[j], sem)
        return carry
    lax.fori_loop(0, KPERW, fire, 0)

    def drain(j, carry):
        pltpu.make_async_copy(table.at[idx_v.at[j]], rows_v.at[j], sem).wait()
        return carry
    lax.fori_loop(0, KPERW, drain, 0)
    pltpu.sync_copy(rows_v, xg_out.at[pl.ds(wid * KPERW, KPERW)])


def _sc_scatter_body(dst2d, msg_in, zeros, acc_out, didx_v, rows_v, acc_sh):
    c = lax.axis_index("c")
    s = lax.axis_index("s")
    wid = s * NC + c
    pltpu.sync_copy(zeros.at[pl.ds(s * STRIPE, STRIPE)],
                    acc_sh.at[pl.ds(s * STRIPE, STRIPE)])
    pltpu.sync_copy(dst2d.at[pl.ds(wid * KPERW, KPERW)], didx_v)
    pltpu.sync_copy(msg_in.at[pl.ds(wid * KPERW, KPERW)], rows_v)
    plsc.subcore_barrier()

    def scat(j, carry):
        pltpu.sync_copy(rows_v.at[j], acc_sh.at[didx_v.at[j]], add=True)
        return carry
    lax.fori_loop(0, KPERW, scat, 0)

    plsc.subcore_barrier()
    pltpu.sync_copy(acc_sh.at[pl.ds(s * STRIPE, STRIPE)],
                    acc_out.at[c, pl.ds(s * STRIPE, STRIPE)])


@functools.cache
def _sc_kernels():
    mesh = plsc.VectorSubcoreMesh(
        core_axis_name="c", subcore_axis_name="s",
        num_cores=NC, num_subcores=NS)
    params = pltpu.CompilerParams(use_tc_tiling_on_sc=False)
    gather_cnt = pl.kernel(
        _sc_gather_cnt_body,
        out_type=[
            jax.ShapeDtypeStruct((EROWS, CHUNK, 16), _f32),
            jax.ShapeDtypeStruct((NC, NACC, 16), _f32),
        ],
        mesh=mesh,
        scratch_types=[
            pltpu.VMEM((KPERW, CHUNK), jnp.int32),
            pltpu.VMEM((KPERW, CHUNK), jnp.int32),
            pltpu.VMEM((KPERW, CHUNK, 16), _f32),
            pltpu.VMEM((CHUNK, 16), _f32),
            pltpu.VMEM_SHARED((NACC, 16), _f32),
            pltpu.SemaphoreType.DMA,
        ],
        compiler_params=params,
    )
    gather = pl.kernel(
        _sc_gather_body,
        out_type=jax.ShapeDtypeStruct((EROWS, CHUNK, 16), _f32),
        mesh=mesh,
        scratch_types=[
            pltpu.VMEM((KPERW, CHUNK), jnp.int32),
            pltpu.VMEM((KPERW, CHUNK, 16), _f32),
            pltpu.SemaphoreType.DMA,
        ],
        compiler_params=params,
    )
    scatter = pl.kernel(
        _sc_scatter_body,
        out_type=jax.ShapeDtypeStruct((NC, NACC, 16), _f32),
        mesh=mesh,
        scratch_types=[
            pltpu.VMEM((KPERW, CHUNK), jnp.int32),
            pltpu.VMEM((KPERW, CHUNK, 16), _f32),
            pltpu.VMEM_SHARED((NACC, 16), _f32),
        ],
        compiler_params=params,
    )
    return gather_cnt, gather, scatter


BE = 2048
BN = 2048


_HI = jax.lax.Precision.HIGHEST


def _edge_mlp_body(ea_ref, xg_ref, e1w_ref, e1b_ref, e2w_ref, e2b_ref,
                   out_ref):
    ht = jnp.dot(e1w_ref[...], ea_ref[...], preferred_element_type=_f32)
    ht = _leaky(ht + e1b_ref[...])
    wft = jnp.dot(e2w_ref[...], ht, preferred_element_type=_f32)
    wft = wft + e2b_ref[...]
    p = wft.reshape(16, 16, BE) * xg_ref[...][:, None, :]
    out_ref[...] = jnp.sum(p, axis=0)


def _edge_mlp(ea_t, xg_t, e1w_t, e1b_t, e2w_t, e2b_t):
    grid = EPAD // BE
    return pl.pallas_call(
        _edge_mlp_body,
        grid=(grid,),
        in_specs=[
            pl.BlockSpec((16, BE), lambda i: (0, i)),
            pl.BlockSpec((16, BE), lambda i: (0, i)),
            pl.BlockSpec((D, 16), lambda i: (0, 0)),
            pl.BlockSpec((D, 1), lambda i: (0, 0)),
            pl.BlockSpec((D, D), lambda i: (0, 0)),
            pl.BlockSpec((D, 1), lambda i: (0, 0)),
        ],
        out_specs=pl.BlockSpec((16, BE), lambda i: (0, i)),
        out_shape=jax.ShapeDtypeStruct((16, EPAD), _f32),
    )(ea_t, xg_t, e1w_t, e1b_t, e2w_t, e2b_t)


def _aggr_body(acc_ref, cnt_ref, x_ref, root_ref, bias_ref, out_ref, *, leaky):
    a = acc_ref[...]
    s = a[0] + a[1]
    ca = cnt_ref[...]
    cnt = ca[0, :, :1] + ca[1, :, :1]
    r = s / jnp.maximum(cnt, 1.0)
    r = r + jnp.dot(x_ref[...], root_ref[...], preferred_element_type=_f32)
    r = r + bias_ref[...]
    if leaky:
        r = _leaky(r)
    out_ref[...] = r


def _aggr(acc, cnt, x, root, bias, leaky):
    grid = NACC // BN
    return pl.pallas_call(
        functools.partial(_aggr_body, leaky=leaky),
        grid=(grid,),
        in_specs=[
            pl.BlockSpec((NC, BN, 16), lambda i: (0, i, 0)),
            pl.BlockSpec((NC, BN, 16), lambda i: (0, i, 0)),
            pl.BlockSpec((BN, 16), lambda i: (i, 0)),
            pl.BlockSpec((16, 16), lambda i: (0, 0)),
            pl.BlockSpec((1, 16), lambda i: (0, 0)),
        ],
        out_specs=pl.BlockSpec((BN, 16), lambda i: (i, 0)),
        out_shape=jax.ShapeDtypeStruct((NACC, 16), _f32),
    )(acc, cnt, x, root, bias)


def _pool_body(h_ref, w_ref, b_ref, day_ref, sec_ref, m1a_ref, m1d_ref,
               m1s_ref, m1b_ref, m2w_ref, m2b_ref, m3w_ref, m3b_ref,
               out_ref, acc_s, cnt_s):
    i = pl.program_id(0)

    @pl.when(i == 0)
    def _():
        acc_s[...] = jnp.zeros_like(acc_s)
        cnt_s[...] = jnp.zeros_like(cnt_s)

    oh = (b_ref[...] == lax.broadcasted_iota(jnp.int32, (BN, NG), 1))
    oh = oh.astype(_f32)
    w = w_ref[...]
    hv = h_ref[...] * w
    dn = (((0,), (0,)), ((), ()))
    acc_s[...] += lax.dot_general(oh, hv, dn, preferred_element_type=_f32,
                                  precision=_HI)
    cnt_s[...] += lax.dot_general(oh, w, dn, preferred_element_type=_f32,
                                  precision=_HI)

    @pl.when(i == (NACC // BN) - 1)
    def _():
        pooled = acc_s[...] / jnp.maximum(cnt_s[...], 1.0)
        z = jnp.dot(pooled, m1a_ref[...], preferred_element_type=_f32)
        z = z + day_ref[...] * m1d_ref[...] + sec_ref[...] * m1s_ref[...]
        z = _leaky(z + m1b_ref[...])
        z = _leaky(jnp.dot(z, m2w_ref[...], preferred_element_type=_f32)
                   + m2b_ref[...])
        out_ref[...] = (jnp.dot(z, m3w_ref[...], preferred_element_type=_f32)
                        + m3b_ref[...])


def _pool(h, w, b, day, sec, m1a, m1d, m1s, m1b, m2w, m2b, m3w, m3b):
    grid = NACC // BN
    return pl.pallas_call(
        _pool_body,
        grid=(grid,),
        in_specs=[
            pl.BlockSpec((BN, 16), lambda i: (i, 0)),
            pl.BlockSpec((BN, 1), lambda i: (i, 0)),
            pl.BlockSpec((BN, 1), lambda i: (i, 0)),
            pl.BlockSpec((NG, 1), lambda i: (0, 0)),
            pl.BlockSpec((NG, 1), lambda i: (0, 0)),
            pl.BlockSpec((16, 128), lambda i: (0, 0)),
            pl.BlockSpec((1, 128), lambda i: (0, 0)),
            pl.BlockSpec((1, 128), lambda i: (0, 0)),
            pl.BlockSpec((1, 128), lambda i: (0, 0)),
            pl.BlockSpec((128, 128), lambda i: (0, 0)),
            pl.BlockSpec((1, 128), lambda i: (0, 0)),
            pl.BlockSpec((128, 1), lambda i: (0, 0)),
            pl.BlockSpec((1, 1), lambda i: (0, 0)),
        ],
        out_specs=pl.BlockSpec((NG, 1), lambda i: (0, 0)),
        out_shape=jax.ShapeDtypeStruct((NG, 1), _f32),
        scratch_shapes=[
            pltpu.VMEM((NG, 16), _f32),
            pltpu.VMEM((NG, 1), _f32),
        ],
    )(h, w, b, day, sec, m1a, m1d, m1s, m1b, m2w, m2b, m3w, m3b)


@jax.jit
def kernel(x, edge_index, edge_attr, trip_mask, batch, day, sec,
           c1_e1w, c1_e1b, c1_e2w, c1_e2b, c1_root, c1_bias,
           c2_e1w, c2_e1b, c2_e2w, c2_e2b, c2_root, c2_bias,
           m1w, m1b, m2w, m2b, m3w, m3b):
    src = edge_index[0].astype(jnp.int32)
    dst = edge_index[1].astype(jnp.int32)
    src2d = jnp.concatenate(
        [src, jnp.zeros((EPAD - E,), jnp.int32)]).reshape(EROWS, CHUNK)
    dst2d = jnp.concatenate(
        [dst, jnp.full((EPAD - E,), TRASH, jnp.int32)]).reshape(EROWS, CHUNK)
    ea_p = jnp.concatenate(
        [edge_attr, jnp.zeros((EPAD - E, EA), _f32)], axis=0)
    x_p = jnp.concatenate([x, jnp.zeros((NACC - N, IN), _f32)], axis=0)
    zeros_acc = jnp.zeros((NACC, 16), _f32)
    ones_chunk = jnp.ones((CHUNK, 16), _f32)
    _gather_cnt, _gather, _scatter = _sc_kernels()

    ea_t = ea_p.T

    xg3, cntp = _gather_cnt(src2d, dst2d, x_p, zeros_acc, ones_chunk)
    msg1_t = _edge_mlp(ea_t, xg3.reshape(EPAD, 16).T,
                       c1_e1w.T, c1_e1b.reshape(D, 1), c1_e2w.T,
                       c1_e2b.reshape(D, 1))
    acc1 = _scatter(dst2d, msg1_t.T.reshape(EROWS, CHUNK, 16), zeros_acc)
    h1 = _aggr(acc1, cntp, x_p, c1_root, c1_bias.reshape(1, 16), leaky=True)

    hg3 = _gather(src2d, h1)
    msg2_t = _edge_mlp(ea_t, hg3.reshape(EPAD, 16).T,
                       c2_e1w.T, c2_e1b.reshape(D, 1), c2_e2w.T,
                       c2_e2b.reshape(D, 1))
    acc2 = _scatter(dst2d, msg2_t.T.reshape(EROWS, CHUNK, 16), zeros_acc)
    h2 = _aggr(acc2, cntp, h1, c2_root, c2_bias.reshape(1, 16), leaky=False)

    w_p = jnp.concatenate(
        [trip_mask.astype(_f32), jnp.zeros((NACC - N,), _f32)]).reshape(NACC, 1)
    b_p = jnp.concatenate(
        [batch.astype(jnp.int32),
         jnp.full((NACC - N,), NG, jnp.int32)]).reshape(NACC, 1)
    return _pool(h2, w_p, b_p, day.reshape(NG, 1), sec.reshape(NG, 1),
                 m1w[:16], m1w[16:17], m1w[17:18], m1b.reshape(1, 128),
                 m2w, m2b.reshape(1, 128), m3w, m3b.reshape(1, 1))
